# baseline (device time: 382063 ns/iter reference)
import jax
import jax.numpy as jnp
from jax import lax
from jax.experimental import pallas as pl
from jax.experimental.pallas import tpu as pltpu

N_DEV = 8
S = 1024
H = 8
DH = 128
BLK = 64
SCALE = 0.08838834764831843


def _attn_body(q_ref, k_ref, v_ref, mask_ref, out_ref, qbuf, pacc,
               qstage, cstage, accbuf, mlbuf, pml,
               m_ref, l_ref, acc_ref, mx_ref,
               q_send_sems, q_recv_sems, p_send_sems, p_recv_sems, copy_sem):
    my = lax.axis_index("i")
    f32 = jnp.float32
    bf = jnp.bfloat16

    def q_rdma(target):
        return pltpu.make_async_remote_copy(
            src_ref=q_ref,
            dst_ref=qbuf.at[my],
            send_sem=q_send_sems.at[target],
            recv_sem=q_recv_sems.at[my],
            device_id=(target,),
            device_id_type=pl.DeviceIdType.MESH,
        )

    def q_recv(source):
        return pltpu.make_async_remote_copy(
            src_ref=q_ref,
            dst_ref=qbuf.at[source],
            send_sem=q_send_sems.at[source],
            recv_sem=q_recv_sems.at[source],
            device_id=(my,),
            device_id_type=pl.DeviceIdType.MESH,
        )

    def p_rdma(slot, target):
        acc = pltpu.make_async_remote_copy(
            src_ref=accbuf.at[slot],
            dst_ref=pacc.at[my],
            send_sem=p_send_sems.at[target, 0],
            recv_sem=p_recv_sems.at[my, 0],
            device_id=(target,),
            device_id_type=pl.DeviceIdType.MESH,
        )
        ml = pltpu.make_async_remote_copy(
            src_ref=mlbuf.at[slot],
            dst_ref=pml.at[my],
            send_sem=p_send_sems.at[target, 1],
            recv_sem=p_recv_sems.at[my, 1],
            device_id=(target,),
            device_id_type=pl.DeviceIdType.MESH,
        )
        return acc, ml

    def p_recv(source, which):
        dst = pacc.at[source] if which == 0 else pml.at[source]
        src = accbuf.at[0] if which == 0 else mlbuf.at[0]
        return pltpu.make_async_remote_copy(
            src_ref=src,
            dst_ref=dst,
            send_sem=p_send_sems.at[source, which],
            recv_sem=p_recv_sems.at[source, which],
            device_id=(my,),
            device_id_type=pl.DeviceIdType.MESH,
        )

    def partial(qh, kh, vh, bias):
        sT = lax.dot_general(
            kh, qh, (((1,), (1,)), ((), ())),
            preferred_element_type=f32,
        )
        if bias is not None:
            sT = sT + bias
        m = jnp.max(sT, axis=0, keepdims=True)
        p = jnp.exp(sT - m)
        l = jnp.sum(p, axis=0, keepdims=True)
        accT = lax.dot_general(
            vh, p.astype(bf), (((0,), (0,)), ((), ())),
            preferred_element_type=f32,
        )
        return m, l, accT

    for d in range(1, N_DEV):
        @pl.when(my >= d)
        def _():
            q_rdma(my - d).start()

    for d in range(N_DEV - 1, 0, -1):
        @pl.when(my + d <= N_DEV - 1)
        def _():
            src = my + d
            q_recv(src).wait_recv()
            cp = pltpu.make_async_copy(qbuf.at[src], qstage, copy_sem)
            cp.start()
            cp.wait()

            if d + 2 <= N_DEV - 1:
                @pl.when(my + d + 2 <= N_DEV - 1)
                def _():
                    a, m = p_rdma(d % 2, my + d + 2)
                    a.wait_send()
                    m.wait_send()

            def head_partial(h, _):
                m, l, accT = partial(qstage[h], k_ref[h], v_ref[h], None)
                accbuf[d % 2, h] = accT.astype(bf)
                mlbuf[d % 2, 0, h] = m.astype(bf)
                mlbuf[d % 2, 1, h] = l.astype(bf)
                return 0

            lax.fori_loop(0, H, head_partial, 0)

            a, m = p_rdma(d % 2, src)
            a.start()
            m.start()

    def diag_head(h, _):
        m, l, accT = partial(q_ref[h], k_ref[h], v_ref[h],
                             mask_ref[...].astype(f32))
        m_ref[h] = m
        l_ref[h] = l
        acc_ref[h] = accT
        return 0

    lax.fori_loop(0, H, diag_head, 0)

    mx_ref[...] = m_ref[...]
    for e in range(N_DEV - 1, 0, -1):
        @pl.when(my >= e)
        def _():
            p_recv(my - e, 1).wait_recv()
            mx_ref[...] = jnp.maximum(mx_ref[...], pml[my - e, 0].astype(f32))

    w_own = jnp.exp(m_ref[...] - mx_ref[...])
    l_ref[...] = l_ref[...] * w_own
    acc_ref[...] = acc_ref[...] * w_own

    for e in range(N_DEV - 1, 0, -1):
        @pl.when(my >= e)
        def _():
            src = my - e
            p_recv(src, 0).wait_recv()
            cp = pltpu.make_async_copy(pacc.at[src], cstage, copy_sem)
            cp.start()
            cp.wait()
            w = jnp.exp(pml[src, 0].astype(f32) - mx_ref[...])
            acc_ref[...] = acc_ref[...] + w * cstage[...].astype(f32)
            l_ref[...] = l_ref[...] + w * pml[src, 1].astype(f32)

    out_ref[...] = (acc_ref[...] / l_ref[...]).astype(bf)

    for d in range(1, N_DEV):
        @pl.when(my >= d)
        def _():
            q_rdma(my - d).wait_send()
    for d in [1, 2]:
        @pl.when(my + d <= N_DEV - 1)
        def _():
            a, m = p_rdma(d % 2, my + d)
            a.wait_send()
            m.wait_send()


def _attn(q3, k3, v3, mask_t):
    ctx, _, _ = pl.pallas_call(
        _attn_body,
        out_shape=[
            jax.ShapeDtypeStruct((H, DH, S), jnp.bfloat16),
            jax.ShapeDtypeStruct((N_DEV, H, S, DH), jnp.bfloat16),
            jax.ShapeDtypeStruct((N_DEV, H, DH, S), jnp.bfloat16),
        ],
        in_specs=[pl.BlockSpec(memory_space=pltpu.VMEM)] * 4,
        out_specs=[
            pl.BlockSpec(memory_space=pltpu.VMEM),
            pl.BlockSpec(memory_space=pltpu.MemorySpace.HBM),
            pl.BlockSpec(memory_space=pltpu.MemorySpace.HBM),
        ],
        scratch_shapes=[
            pltpu.VMEM((H, S, DH), jnp.bfloat16),
            pltpu.VMEM((H, DH, S), jnp.bfloat16),
            pltpu.VMEM((2, H, DH, S), jnp.bfloat16),
            pltpu.VMEM((2, 2, H, 1, S), jnp.bfloat16),
            pltpu.VMEM((N_DEV, 2, H, 1, S), jnp.bfloat16),
            pltpu.VMEM((H, 1, S), jnp.float32),
            pltpu.VMEM((H, 1, S), jnp.float32),
            pltpu.VMEM((H, DH, S), jnp.float32),
            pltpu.VMEM((H, 1, S), jnp.float32),
            pltpu.SemaphoreType.DMA((N_DEV,)),
            pltpu.SemaphoreType.DMA((N_DEV,)),
            pltpu.SemaphoreType.DMA((N_DEV, 2)),
            pltpu.SemaphoreType.DMA((N_DEV, 2)),
            pltpu.SemaphoreType.DMA,
        ],
    )(q3, k3, v3, mask_t)
    return ctx


def kernel(x, Wq, K_ext, V_ext, Wo):
    bf = jnp.bfloat16
    q = (x[0].astype(bf) @ Wq.astype(bf)).astype(jnp.float32) * SCALE
    q3 = q.reshape(S, H, DH).transpose(1, 0, 2).astype(bf)
    k3 = K_ext[0].transpose(1, 0, 2).astype(bf)
    v3 = V_ext[0].transpose(1, 0, 2).astype(bf)
    qb = jnp.arange(S)[None, :] // BLK
    kb = jnp.arange(S)[:, None] // BLK
    mask_t = jnp.where(kb > qb, -1e9, 0.0).astype(bf)
    ctx = _attn(q3, k3, v3, mask_t)
    ctx2 = ctx.transpose(2, 0, 1).reshape(S, H * DH)
    out = ctx2 @ Wo.astype(bf)
    return out.astype(jnp.float32)[None]


# device time: 242526 ns/iter; 1.5753x vs baseline; 1.5753x over previous
import jax
import jax.numpy as jnp
from jax import lax
from jax.experimental import pallas as pl
from jax.experimental.pallas import tpu as pltpu

N_DEV = 8
S = 1024
H = 8
DH = 128
BLK = 64
SCALE = 0.08838834764831843


def _attn_body(q_ref, k_ref, v_ref, mask_ref, out_ref, qbuf, pacc,
               qstage, cstage, accbuf, mlbuf, pml,
               m_ref, l_ref, acc_ref, mx_ref,
               q_send_sems, q_recv_sems, p_send_sems, p_recv_sems, copy_sem):
    my = lax.axis_index("i")
    f32 = jnp.float32
    bf = jnp.bfloat16

    def q_rdma(target):
        return pltpu.make_async_remote_copy(
            src_ref=q_ref,
            dst_ref=qbuf.at[my],
            send_sem=q_send_sems.at[target],
            recv_sem=q_recv_sems.at[my],
            device_id=(target,),
            device_id_type=pl.DeviceIdType.MESH,
        )

    def q_recv(source):
        return pltpu.make_async_remote_copy(
            src_ref=q_ref,
            dst_ref=qbuf.at[source],
            send_sem=q_send_sems.at[source],
            recv_sem=q_recv_sems.at[source],
            device_id=(my,),
            device_id_type=pl.DeviceIdType.MESH,
        )

    def p_rdma(slot, target):
        acc = pltpu.make_async_remote_copy(
            src_ref=accbuf.at[slot],
            dst_ref=pacc.at[my],
            send_sem=p_send_sems.at[target, 0],
            recv_sem=p_recv_sems.at[my, 0],
            device_id=(target,),
            device_id_type=pl.DeviceIdType.MESH,
        )
        ml = pltpu.make_async_remote_copy(
            src_ref=mlbuf.at[slot],
            dst_ref=pml.at[my],
            send_sem=p_send_sems.at[target, 1],
            recv_sem=p_recv_sems.at[my, 1],
            device_id=(target,),
            device_id_type=pl.DeviceIdType.MESH,
        )
        return acc, ml

    def p_recv(source, which):
        dst = pacc.at[source] if which == 0 else pml.at[source]
        src = accbuf.at[0] if which == 0 else mlbuf.at[0]
        return pltpu.make_async_remote_copy(
            src_ref=src,
            dst_ref=dst,
            send_sem=p_send_sems.at[source, which],
            recv_sem=p_recv_sems.at[source, which],
            device_id=(my,),
            device_id_type=pl.DeviceIdType.MESH,
        )

    def partial(qh, kh, vh, bias):
        sT = lax.dot_general(
            kh, qh, (((1,), (1,)), ((), ())),
            preferred_element_type=f32,
        )
        if bias is not None:
            sT = sT + bias
        m = jnp.max(sT, axis=0, keepdims=True)
        p = jnp.exp(sT - m)
        l = jnp.sum(p, axis=0, keepdims=True)
        accT = lax.dot_general(
            vh, p.astype(bf), (((0,), (0,)), ((), ())),
            preferred_element_type=f32,
        )
        return m, l, accT

    for d in range(1, N_DEV):
        @pl.when(my >= d)
        def _():
            q_rdma(my - d).start()

    for d in range(1, N_DEV):
        @pl.when(my + d <= N_DEV - 1)
        def _():
            src = my + d
            q_recv(src).wait_recv()
            cp = pltpu.make_async_copy(qbuf.at[src], qstage, copy_sem)
            cp.start()
            cp.wait()

            if d - 2 >= 1:
                a, m = p_rdma(d % 2, my + d - 2)
                a.wait_send()
                m.wait_send()

            def head_partial(h, _):
                m, l, accT = partial(qstage[h], k_ref[h], v_ref[h], None)
                accbuf[d % 2, h] = accT.astype(bf)
                mlbuf[d % 2, 0, h] = m.astype(bf)
                mlbuf[d % 2, 1, h] = l.astype(bf)
                return 0

            lax.fori_loop(0, H, head_partial, 0)

            a, m = p_rdma(d % 2, src)
            a.start()
            m.start()

    def diag_head(h, _):
        m, l, accT = partial(q_ref[h], k_ref[h], v_ref[h],
                             mask_ref[...].astype(f32))
        m_ref[h] = m
        l_ref[h] = l
        acc_ref[h] = accT
        return 0

    lax.fori_loop(0, H, diag_head, 0)

    mx_ref[...] = m_ref[...]
    for e in range(1, N_DEV):
        @pl.when(my >= e)
        def _():
            p_recv(my - e, 1).wait_recv()
            mx_ref[...] = jnp.maximum(mx_ref[...], pml[my - e, 0].astype(f32))

    w_own = jnp.exp(m_ref[...] - mx_ref[...])
    l_ref[...] = l_ref[...] * w_own
    acc_ref[...] = acc_ref[...] * w_own

    for e in range(1, N_DEV):
        @pl.when(my >= e)
        def _():
            src = my - e
            p_recv(src, 0).wait_recv()
            cp = pltpu.make_async_copy(pacc.at[src], cstage, copy_sem)
            cp.start()
            cp.wait()
            w = jnp.exp(pml[src, 0].astype(f32) - mx_ref[...])
            acc_ref[...] = acc_ref[...] + w * cstage[...].astype(f32)
            l_ref[...] = l_ref[...] + w * pml[src, 1].astype(f32)

    out_ref[...] = (acc_ref[...] / l_ref[...]).astype(bf)

    for d in range(1, N_DEV):
        @pl.when(my >= d)
        def _():
            q_rdma(my - d).wait_send()
    @pl.when(my < N_DEV - 1)
    def _():
        a, m = p_rdma((N_DEV - 1 - my) % 2, N_DEV - 1)
        a.wait_send()
        m.wait_send()

    @pl.when(my < N_DEV - 2)
    def _():
        a, m = p_rdma((N_DEV - 2 - my) % 2, N_DEV - 2)
        a.wait_send()
        m.wait_send()


def _attn(q3, k3, v3, mask_t):
    ctx, _, _ = pl.pallas_call(
        _attn_body,
        out_shape=[
            jax.ShapeDtypeStruct((H, DH, S), jnp.bfloat16),
            jax.ShapeDtypeStruct((N_DEV, H, S, DH), jnp.bfloat16),
            jax.ShapeDtypeStruct((N_DEV, H, DH, S), jnp.bfloat16),
        ],
        in_specs=[pl.BlockSpec(memory_space=pltpu.VMEM)] * 4,
        out_specs=[
            pl.BlockSpec(memory_space=pltpu.VMEM),
            pl.BlockSpec(memory_space=pltpu.MemorySpace.HBM),
            pl.BlockSpec(memory_space=pltpu.MemorySpace.HBM),
        ],
        scratch_shapes=[
            pltpu.VMEM((H, S, DH), jnp.bfloat16),
            pltpu.VMEM((H, DH, S), jnp.bfloat16),
            pltpu.VMEM((2, H, DH, S), jnp.bfloat16),
            pltpu.VMEM((2, 2, H, 1, S), jnp.bfloat16),
            pltpu.VMEM((N_DEV, 2, H, 1, S), jnp.bfloat16),
            pltpu.VMEM((H, 1, S), jnp.float32),
            pltpu.VMEM((H, 1, S), jnp.float32),
            pltpu.VMEM((H, DH, S), jnp.float32),
            pltpu.VMEM((H, 1, S), jnp.float32),
            pltpu.SemaphoreType.DMA((N_DEV,)),
            pltpu.SemaphoreType.DMA((N_DEV,)),
            pltpu.SemaphoreType.DMA((N_DEV, 2)),
            pltpu.SemaphoreType.DMA((N_DEV, 2)),
            pltpu.SemaphoreType.DMA,
        ],
    )(q3, k3, v3, mask_t)
    return ctx


def kernel(x, Wq, K_ext, V_ext, Wo):
    bf = jnp.bfloat16
    q = (x[0].astype(bf) @ Wq.astype(bf)).astype(jnp.float32) * SCALE
    q3 = q.reshape(S, H, DH).transpose(1, 0, 2).astype(bf)
    k3 = K_ext[0].transpose(1, 0, 2).astype(bf)
    v3 = V_ext[0].transpose(1, 0, 2).astype(bf)
    qb = jnp.arange(S)[None, :] // BLK
    kb = jnp.arange(S)[:, None] // BLK
    mask_t = jnp.where(kb > qb, -1e9, 0.0).astype(bf)
    ctx = _attn(q3, k3, v3, mask_t)
    ctx2 = ctx.transpose(2, 0, 1).reshape(S, H * DH)
    out = ctx2 @ Wo.astype(bf)
    return out.astype(jnp.float32)[None]


# device time: 211302 ns/iter; 1.8081x vs baseline; 1.1478x over previous
import jax
import jax.numpy as jnp
from jax import lax
from jax.experimental import pallas as pl
from jax.experimental.pallas import tpu as pltpu

N_DEV = 8
S = 1024
H = 8
DH = 128
BLK = 64
SCALE = 0.08838834764831843


def _attn_body(q_ref, k_ref, v_ref, mask_ref, out_ref, kv_hbm,
               stage_ref, m_ref, l_ref, acc_ref,
               send_sems, recv_sems, copy_sem):
    my = lax.axis_index("i")

    def hop1_rdma(half, target):
        src = k_ref if half == 0 else v_ref
        return pltpu.make_async_remote_copy(
            src_ref=src,
            dst_ref=kv_hbm.at[0, half],
            send_sem=send_sems.at[1, half],
            recv_sem=recv_sems.at[1, half],
            device_id=(target,),
            device_id_type=pl.DeviceIdType.MESH,
        )

    def fwd_rdma(h, target):
        return pltpu.make_async_remote_copy(
            src_ref=kv_hbm.at[h - 2],
            dst_ref=kv_hbm.at[h - 1],
            send_sem=send_sems.at[h, 0],
            recv_sem=recv_sems.at[h, 0],
            device_id=(target,),
            device_id_type=pl.DeviceIdType.MESH,
        )

    def flash_update(h, kh, vh, s_bias):
        qh = q_ref[h]
        s = lax.dot_general(
            qh, kh, (((1,), (1,)), ((), ())),
            preferred_element_type=jnp.float32,
        )
        if s_bias is not None:
            s = s + s_bias
        m = m_ref[h]
        m_new = jnp.maximum(m, jnp.max(s, axis=1, keepdims=True))
        p = jnp.exp(s - m_new)
        alpha = jnp.exp(m - m_new)
        pv = lax.dot_general(
            p.astype(jnp.bfloat16), vh, (((1,), (0,)), ((), ())),
            preferred_element_type=jnp.float32,
        )
        m_ref[h] = m_new
        l_ref[h] = l_ref[h] * alpha + jnp.sum(p, axis=1, keepdims=True)
        acc_ref[h] = acc_ref[h] * alpha + pv

    @pl.when(my < N_DEV - 1)
    def _():
        hop1_rdma(0, my + 1).start()
        hop1_rdma(1, my + 1).start()

    def diag_head(h, _):
        m_ref[h] = jnp.full((S, 1), -1e30, jnp.float32)
        l_ref[h] = jnp.zeros((S, 1), jnp.float32)
        acc_ref[h] = jnp.zeros((S, DH), jnp.float32)
        flash_update(h, k_ref[h], v_ref[h], mask_ref[...].astype(jnp.float32))
        return 0

    lax.fori_loop(0, H, diag_head, 0)

    for h in range(1, N_DEV):
        @pl.when(my >= h)
        def _():
            if h == 1:
                hop1_rdma(0, my).wait_recv()
                hop1_rdma(1, my).wait_recv()
            else:
                fwd_rdma(h, my - 1).wait_recv()

        if h + 1 < N_DEV:
            @pl.when((my < N_DEV - 1) & (my >= h))
            def _():
                fwd_rdma(h + 1, my + 1).start()

        @pl.when(my >= h)
        def _():
            cp = pltpu.make_async_copy(kv_hbm.at[h - 1], stage_ref, copy_sem)
            cp.start()
            cp.wait()

            def head_step(hh, _):
                flash_update(hh, stage_ref[0, hh], stage_ref[1, hh], None)
                return 0

            lax.fori_loop(0, H, head_step, 0)

    @pl.when(my < N_DEV - 1)
    def _():
        hop1_rdma(0, my + 1).wait_send()
        hop1_rdma(1, my + 1).wait_send()
    for h in range(2, N_DEV):
        @pl.when((my < N_DEV - 1) & (my >= h - 1))
        def _():
            fwd_rdma(h, my + 1).wait_send()

    def finish(h, _):
        out_ref[h] = (acc_ref[h] / l_ref[h]).astype(jnp.bfloat16)
        return 0

    lax.fori_loop(0, H, finish, 0)


def _attn(q3, k3, v3, mask_add):
    ctx, _ = pl.pallas_call(
        _attn_body,
        out_shape=[
            jax.ShapeDtypeStruct((H, S, DH), jnp.bfloat16),
            jax.ShapeDtypeStruct((N_DEV - 1, 2, H, S, DH), jnp.bfloat16),
        ],
        in_specs=[pl.BlockSpec(memory_space=pltpu.VMEM)] * 4,
        out_specs=[
            pl.BlockSpec(memory_space=pltpu.VMEM),
            pl.BlockSpec(memory_space=pltpu.MemorySpace.HBM),
        ],
        scratch_shapes=[
            pltpu.VMEM((2, H, S, DH), jnp.bfloat16),
            pltpu.VMEM((H, S, 1), jnp.float32),
            pltpu.VMEM((H, S, 1), jnp.float32),
            pltpu.VMEM((H, S, DH), jnp.float32),
            pltpu.SemaphoreType.DMA((N_DEV, 2)),
            pltpu.SemaphoreType.DMA((N_DEV, 2)),
            pltpu.SemaphoreType.DMA,
        ],
    )(q3, k3, v3, mask_add)
    return ctx


def kernel(x, Wq, K_ext, V_ext, Wo):
    bf = jnp.bfloat16
    q = (x[0].astype(bf) @ Wq.astype(bf)).astype(jnp.float32) * SCALE
    q3 = q.reshape(S, H, DH).transpose(1, 0, 2).astype(bf)
    k3 = K_ext[0].transpose(1, 0, 2).astype(bf)
    v3 = V_ext[0].transpose(1, 0, 2).astype(bf)
    row_blk = jnp.arange(S)[:, None] // BLK
    col_blk = jnp.arange(S)[None, :] // BLK
    mask_add = jnp.where(col_blk > row_blk, -1e9, 0.0).astype(bf)
    ctx = _attn(q3, k3, v3, mask_add)
    ctx2 = ctx.transpose(1, 0, 2).reshape(S, H * DH)
    out = ctx2 @ Wo.astype(bf)
    return out.astype(jnp.float32)[None]
